# baseline (speedup 1.0000x reference)
import functools

import jax
import jax.numpy as jnp
from jax.experimental import pallas as pl

K_NN = 10


def _body(xb_ref, xa_ref, h_ref, *, n_rows, blk):
    xb = xb_ref[...]
    xa = xa_ref[...]
    d = xb.shape[1]

    sqa = jnp.sum(xa * xa, axis=1, keepdims=True)
    sqb_row = jax.lax.dot_general(
        jnp.ones((1, d), jnp.float32), xb * xb,
        (((1,), (1,)), ((), ())), precision=jax.lax.Precision.HIGHEST,
        preferred_element_type=jnp.float32)
    mm = jax.lax.dot_general(
        xa, xb, (((1,), (1,)), ((), ())),
        preferred_element_type=jnp.float32)
    dist = jnp.maximum(sqa + sqb_row - 2.0 * mm, 0.0)

    avg = jnp.sum(dist, axis=0, keepdims=True) * (1.0 / n_rows)

    rows = jax.lax.broadcasted_iota(jnp.int32, dist.shape, 0)
    hi = jnp.int32(-8192)
    lo = jnp.int32(8191)
    intmax = jnp.int32(2**31 - 1)
    inf32 = jnp.float32(jnp.inf)

    bits = jax.lax.bitcast_convert_type(dist, jnp.int32)
    pk = (bits & hi) | rows

    gsz = 64
    ng = n_rows // gsz
    pk3 = pk.reshape(ng, gsz, blk)
    g1 = jnp.min(pk3, axis=1)
    g2 = jnp.min(jnp.where(pk3 > g1[:, None, :], pk3, intmax), axis=1)
    g3 = jnp.min(jnp.where(pk3 > g2[:, None, :], pk3, intmax), axis=1)
    g4 = jnp.min(jnp.where(pk3 > g3[:, None, :], pk3, intmax), axis=1)

    work = g1
    hc = jnp.zeros((ng, blk), jnp.int32)
    ovf = jnp.zeros((1, blk), jnp.bool_)
    m = jnp.zeros((1, blk), jnp.int32)
    for _ in range(K_NN):
        m = jnp.min(work, axis=0, keepdims=True)
        hit = work == m
        nxt = jnp.where(hc == 0, g2, jnp.where(hc == 1, g3,
                        jnp.where(hc == 2, g4, intmax)))
        ovf = ovf | jnp.any(hit & (hc >= 3), axis=0, keepdims=True)
        work = jnp.where(hit, nxt, work)
        hc = hc + jnp.where(hit, 1, 0)

    def _flat(_):
        mf = jnp.min(pk, axis=0, keepdims=True)
        for _ in range(K_NN - 1):
            mf = jnp.min(jnp.where(pk > mf, pk, intmax), axis=0,
                         keepdims=True)
        return mf

    m = jax.lax.cond(jnp.any(ovf), _flat, lambda _: m, None)
    t_b = m & hi

    trunc = pk & hi
    in_bucket = trunc == t_b
    nless = jnp.sum(jnp.where(trunc < t_b, 1, 0), axis=0, keepdims=True)
    nbucket = jnp.sum(jnp.where(in_bucket, 1, 0), axis=0, keepdims=True)
    take = K_NN - nless

    need = nbucket > take
    vb0 = jnp.where(need, -inf32, inf32)
    jb0 = jnp.where(need, jnp.int32(-1), intmax)
    cnt0 = jnp.where(need, 0, take)

    def _cond(carry):
        _, _, cnt = carry
        return jnp.any(cnt < take)

    def _refine(carry):
        vb, jb, cnt = carry
        act = cnt < take
        lexgt = (dist > vb) | ((dist == vb) & (rows > jb))
        contrib = jnp.where(in_bucket & lexgt, dist, inf32)
        m2 = jnp.min(contrib, axis=0, keepdims=True)
        j2 = jnp.min(jnp.where(contrib == m2, rows, intmax), axis=0,
                     keepdims=True)
        vb = jnp.where(act, m2, vb)
        jb = jnp.where(act, j2, jb)
        return vb, jb, cnt + jnp.where(act, 1, 0)

    vb, jb, _ = jax.lax.while_loop(_cond, _refine, (vb0, jb0, cnt0))

    mask = (trunc < t_b) | (in_bucket &
                            ((dist < vb) | ((dist == vb) & (rows <= jb))))
    inv = 1.0 / (avg * avg + 1e-12)
    h_ref[...] = jnp.where(mask, jnp.exp(-(dist * dist) * inv), 0.0)


def kernel(inputs):
    x = inputs
    n, d = x.shape
    blk = 128
    grid = n // blk
    body = functools.partial(_body, n_rows=n, blk=blk)
    return pl.pallas_call(
        body,
        grid=(grid,),
        in_specs=[
            pl.BlockSpec((blk, d), lambda i: (i, 0)),
            pl.BlockSpec((n, d), lambda i: (0, 0)),
        ],
        out_specs=pl.BlockSpec((n, blk), lambda i: (0, i)),
        out_shape=jax.ShapeDtypeStruct((n, n), jnp.float32),
    )(x, x)


# speedup vs baseline: 1.4591x; 1.4591x over previous
import functools

import jax
import jax.numpy as jnp
from jax.experimental import pallas as pl

K_NN = 10


def _body(xb_ref, xa_ref, h_ref, *, n_rows, blk):
    xb = xb_ref[...]
    xa = xa_ref[...]
    d = xb.shape[1]

    sqa = jnp.sum(xa * xa, axis=1, keepdims=True)
    sqb_row = jax.lax.dot_general(
        jnp.ones((1, d), jnp.float32), xb * xb,
        (((1,), (1,)), ((), ())), precision=jax.lax.Precision.HIGHEST,
        preferred_element_type=jnp.float32)
    mm = jax.lax.dot_general(
        xa, xb, (((1,), (1,)), ((), ())),
        preferred_element_type=jnp.float32)
    dist = jnp.maximum(sqa + sqb_row - 2.0 * mm, 0.0)

    avg = jnp.sum(dist, axis=0, keepdims=True) * (1.0 / n_rows)

    rows = jax.lax.broadcasted_iota(jnp.int32, dist.shape, 0)
    hi = jnp.int32(-8192)
    lo = jnp.int32(8191)
    intmax = jnp.int32(2**31 - 1)
    inf32 = jnp.float32(jnp.inf)

    bits = jax.lax.bitcast_convert_type(dist, jnp.int32)
    pk = (bits & hi) | rows

    gsz = 64
    ng = n_rows // gsz
    g1 = jnp.full((ng, blk), intmax, jnp.int32)
    g2, g3, g4 = g1, g1, g1
    for s in range(gsz):
        xpl = pk[s * ng:(s + 1) * ng, :]
        c = jnp.maximum(g1, xpl)
        g1 = jnp.minimum(g1, xpl)
        c2 = jnp.maximum(g2, c)
        g2 = jnp.minimum(g2, c)
        c3 = jnp.maximum(g3, c2)
        g3 = jnp.minimum(g3, c2)
        g4 = jnp.minimum(g4, c3)

    work = g1
    hc = jnp.zeros((ng, blk), jnp.int32)
    ovf = jnp.zeros((1, blk), jnp.bool_)
    m = jnp.zeros((1, blk), jnp.int32)
    for _ in range(K_NN):
        m = jnp.min(work, axis=0, keepdims=True)
        hit = work == m
        nxt = jnp.where(hc == 0, g2, jnp.where(hc == 1, g3,
                        jnp.where(hc == 2, g4, intmax)))
        ovf = ovf | jnp.any(hit & (hc >= 3), axis=0, keepdims=True)
        work = jnp.where(hit, nxt, work)
        hc = hc + jnp.where(hit, 1, 0)

    def _flat(_):
        mf = jnp.min(pk, axis=0, keepdims=True)
        for _ in range(K_NN - 1):
            mf = jnp.min(jnp.where(pk > mf, pk, intmax), axis=0,
                         keepdims=True)
        return mf

    m = jax.lax.cond(jnp.any(ovf), _flat, lambda _: m, None)
    t_b = m & hi

    trunc = pk & hi
    in_bucket = trunc == t_b
    nless = jnp.sum(jnp.where(trunc < t_b, 1, 0), axis=0, keepdims=True)
    nbucket = jnp.sum(jnp.where(in_bucket, 1, 0), axis=0, keepdims=True)
    take = K_NN - nless

    need = nbucket > take
    vb0 = jnp.where(need, -inf32, inf32)
    jb0 = jnp.where(need, jnp.int32(-1), intmax)
    cnt0 = jnp.where(need, 0, take)

    def _cond(carry):
        _, _, cnt = carry
        return jnp.any(cnt < take)

    def _refine(carry):
        vb, jb, cnt = carry
        act = cnt < take
        lexgt = (dist > vb) | ((dist == vb) & (rows > jb))
        contrib = jnp.where(in_bucket & lexgt, dist, inf32)
        m2 = jnp.min(contrib, axis=0, keepdims=True)
        j2 = jnp.min(jnp.where(contrib == m2, rows, intmax), axis=0,
                     keepdims=True)
        vb = jnp.where(act, m2, vb)
        jb = jnp.where(act, j2, jb)
        return vb, jb, cnt + jnp.where(act, 1, 0)

    vb, jb, _ = jax.lax.while_loop(_cond, _refine, (vb0, jb0, cnt0))

    mask = (trunc < t_b) | (in_bucket &
                            ((dist < vb) | ((dist == vb) & (rows <= jb))))
    inv = 1.0 / (avg * avg + 1e-12)
    h_ref[...] = jnp.where(mask, jnp.exp(-(dist * dist) * inv), 0.0)


def kernel(inputs):
    x = inputs
    n, d = x.shape
    blk = 256
    grid = n // blk
    body = functools.partial(_body, n_rows=n, blk=blk)
    return pl.pallas_call(
        body,
        grid=(grid,),
        in_specs=[
            pl.BlockSpec((blk, d), lambda i: (i, 0)),
            pl.BlockSpec((n, d), lambda i: (0, 0)),
        ],
        out_specs=pl.BlockSpec((n, blk), lambda i: (0, i)),
        out_shape=jax.ShapeDtypeStruct((n, n), jnp.float32),
    )(x, x)


# speedup vs baseline: 1.6172x; 1.1084x over previous
import functools

import jax
import jax.numpy as jnp
from jax.experimental import pallas as pl

K_NN = 10


def _body(xb_ref, xa_ref, h_ref, *, n_rows, blk):
    xb = xb_ref[...]
    xa = xa_ref[...]
    d = xb.shape[1]

    sqa = jnp.sum(xa * xa, axis=1, keepdims=True)
    sqb_row = jax.lax.dot_general(
        jnp.ones((1, d), jnp.float32), xb * xb,
        (((1,), (1,)), ((), ())), precision=jax.lax.Precision.HIGHEST,
        preferred_element_type=jnp.float32)
    mm = jax.lax.dot_general(
        xa, xb, (((1,), (1,)), ((), ())),
        preferred_element_type=jnp.float32)
    dist = jnp.maximum(sqa + sqb_row - 2.0 * mm, 0.0)

    avg = jnp.sum(dist, axis=0, keepdims=True) * (1.0 / n_rows)

    rows = jax.lax.broadcasted_iota(jnp.int32, dist.shape, 0)
    hi = jnp.int32(-8192)
    lo = jnp.int32(8191)
    intmax = jnp.int32(2**31 - 1)
    inf32 = jnp.float32(jnp.inf)

    gsz = 64
    ng = n_rows // gsz
    rows_pl = jax.lax.broadcasted_iota(jnp.int32, (ng, blk), 0)
    g1 = jnp.full((ng, blk), intmax, jnp.int32)
    g2, g3, g4 = g1, g1, g1
    for s in range(gsz):
        bpl = jax.lax.bitcast_convert_type(
            dist[s * ng:(s + 1) * ng, :], jnp.int32)
        xpl = (bpl & hi) | (rows_pl + s * ng)
        c = jnp.maximum(g1, xpl)
        g1 = jnp.minimum(g1, xpl)
        c2 = jnp.maximum(g2, c)
        g2 = jnp.minimum(g2, c)
        c3 = jnp.maximum(g3, c2)
        g3 = jnp.minimum(g3, c2)
        g4 = jnp.minimum(g4, c3)

    work = g1
    hc = jnp.zeros((ng, blk), jnp.int32)
    ovf = jnp.zeros((1, blk), jnp.bool_)
    m = jnp.zeros((1, blk), jnp.int32)
    m2nd = m
    for _ in range(K_NN + 1):
        m, m2nd = m2nd, m
        m2nd = jnp.min(work, axis=0, keepdims=True)
        hit = work == m2nd
        nxt = jnp.where(hc == 0, g2, jnp.where(hc == 1, g3,
                        jnp.where(hc == 2, g4, intmax)))
        ovf = ovf | jnp.any(hit & (hc >= 3), axis=0, keepdims=True)
        work = jnp.where(hit, nxt, work)
        hc = hc + jnp.where(hit, 1, 0)

    def _flat(_):
        bitsf = jax.lax.bitcast_convert_type(dist, jnp.int32)
        pkf = (bitsf & hi) | rows
        mf = jnp.min(pkf, axis=0, keepdims=True)
        for _ in range(K_NN):
            mp = mf
            mf = jnp.min(jnp.where(pkf > mf, pkf, intmax), axis=0,
                         keepdims=True)
        return mp, mf

    m, m2nd = jax.lax.cond(jnp.any(ovf), _flat, lambda _: (m, m2nd), None)
    t_b = m & hi

    trunc = jax.lax.bitcast_convert_type(dist, jnp.int32) & hi
    in_bucket = trunc == t_b
    need = (m2nd & hi) == t_b

    def _nless(_):
        return jnp.sum(jnp.where(trunc < t_b, 1, 0), axis=0, keepdims=True)

    nless = jax.lax.cond(jnp.any(need), _nless,
                         lambda _: jnp.zeros((1, blk), jnp.int32), None)
    take = jnp.where(need, K_NN - nless, 1)

    vb0 = jnp.where(need, -inf32, inf32)
    jb0 = jnp.where(need, jnp.int32(-1), intmax)
    cnt0 = jnp.where(need, 0, take)

    def _cond(carry):
        _, _, cnt = carry
        return jnp.any(cnt < take)

    def _refine(carry):
        vb, jb, cnt = carry
        act = cnt < take
        lexgt = (dist > vb) | ((dist == vb) & (rows > jb))
        contrib = jnp.where(in_bucket & lexgt, dist, inf32)
        m2 = jnp.min(contrib, axis=0, keepdims=True)
        j2 = jnp.min(jnp.where(contrib == m2, rows, intmax), axis=0,
                     keepdims=True)
        vb = jnp.where(act, m2, vb)
        jb = jnp.where(act, j2, jb)
        return vb, jb, cnt + jnp.where(act, 1, 0)

    vb, jb, _ = jax.lax.while_loop(_cond, _refine, (vb0, jb0, cnt0))

    mask = (trunc < t_b) | (in_bucket &
                            ((dist < vb) | ((dist == vb) & (rows <= jb))))
    inv = 1.0 / (avg * avg + 1e-12)
    h_ref[...] = jnp.where(mask, jnp.exp(-(dist * dist) * inv), 0.0)


def kernel(inputs):
    x = inputs
    n, d = x.shape
    blk = 256
    grid = n // blk
    body = functools.partial(_body, n_rows=n, blk=blk)
    return pl.pallas_call(
        body,
        grid=(grid,),
        in_specs=[
            pl.BlockSpec((blk, d), lambda i: (i, 0)),
            pl.BlockSpec((n, d), lambda i: (0, 0)),
        ],
        out_specs=pl.BlockSpec((n, blk), lambda i: (0, i)),
        out_shape=jax.ShapeDtypeStruct((n, n), jnp.float32),
    )(x, x)
